# baseline (speedup 1.0000x reference)
import functools

import jax
import jax.numpy as jnp
from jax import lax
from jax.experimental import pallas as pl
from jax.experimental.pallas import tpu as pltpu
from jax.experimental.pallas import tpu_sc as plsc

B = 16384
D = 64


@functools.cache
def _build_kernel():
    info = plsc.get_sparse_core_info()
    nw = info.num_cores * info.num_subcores
    b_per_w = B // nw
    mesh = plsc.VectorSubcoreMesh(core_axis_name="c", subcore_axis_name="s")

    @functools.partial(
        pl.kernel,
        mesh=mesh,
        out_type=(
            jax.ShapeDtypeStruct((B, D), jnp.float32),
            jax.ShapeDtypeStruct((B, D), jnp.float32),
        ),
        scratch_types=[
            pltpu.VMEM((b_per_w,), jnp.int32),
            pltpu.VMEM((b_per_w, D), jnp.float32),
            pltpu.VMEM((b_per_w, D), jnp.float32),
            pltpu.SemaphoreType.DMA,
            pltpu.SemaphoreType.DMA,
        ],
        compiler_params=pltpu.CompilerParams(use_tc_tiling_on_sc=False),
    )
    def k(ids_hbm, ws_hbm, wa_hbm, out_s_hbm, out_a_hbm,
          idx_v, rows_s, rows_a, sem_s, sem_a):
        wid = lax.axis_index("s") * info.num_cores + lax.axis_index("c")
        base = wid * b_per_w
        pltpu.sync_copy(ids_hbm.at[pl.ds(base, b_per_w)], idx_v)
        cp_s = pltpu.async_copy(ws_hbm.at[idx_v], rows_s, sem_s)
        cp_a = pltpu.async_copy(wa_hbm.at[idx_v], rows_a, sem_a)
        cp_s.wait()
        pltpu.sync_copy(rows_s, out_s_hbm.at[pl.ds(base, b_per_w)])
        cp_a.wait()
        pltpu.sync_copy(rows_a, out_a_hbm.at[pl.ds(base, b_per_w)])

    return k


def kernel(instance_ids, W_shape, W_appearance):
    ids = instance_ids.astype(jnp.int32)
    return _build_kernel()(ids, W_shape, W_appearance)
